# baseline (speedup 1.0000x reference)
import jax
import jax.numpy as jnp
from jax import lax
from jax.experimental import pallas as pl
from jax.experimental.pallas import tpu as pltpu
from jax.experimental.pallas import tpu_sc as plsc

NC = 2
NS = 16
NW = NC * NS
L = 16

SJ = 2
SC_ = 100
S = SJ * SC_
D = 64
KD = D // L

_EPS = 1e-5
_MAGIC = 0x5F3759DF

_DNUMS = lax.GatherDimensionNumbers(
    offset_dims=(), collapsed_slice_dims=(0,), start_index_map=(0,))


def _bcast_lane(vec, lane):
  idx = jnp.full((L, 1), lane, jnp.int32)
  return lax.gather(vec, idx, dimension_numbers=_DNUMS, slice_sizes=(1,),
                    mode=lax.GatherScatterMode.PROMISE_IN_BOUNDS)


def _body(x_hbm, seg_hbm, tok_hbm, pos_hbm, segtab_hbm, g_hbm, b_hbm,
          out_hbm, xidx_v, segv, posv, segtab_v, gv, bv,
          rows0, rows1, ostg0, ostg1, sem_g0, sem_g1, sem_o0, sem_o1):
  nb = x_hbm.shape[0] // NW
  wid = lax.axis_index("s") * NC + lax.axis_index("c")
  b0 = wid * nb

  pltpu.sync_copy(pos_hbm, posv)
  pltpu.sync_copy(segtab_hbm, segtab_v)
  pltpu.sync_copy(g_hbm, gv)
  pltpu.sync_copy(b_hbm, bv)
  pltpu.sync_copy(x_hbm.at[pl.ds(b0, nb)], xidx_v)
  pltpu.sync_copy(seg_hbm.at[pl.ds(b0, nb)], segv)

  g = [gv[k] for k in range(KD)]
  bt = [bv[k] for k in range(KD)]
  s0 = [segtab_v[0, pl.ds(k * L, L)] for k in range(KD)]
  sd = [segtab_v[1, pl.ds(k * L, L)] - s0[k] for k in range(KD)]

  rows_sl = (rows0, rows1)
  ostg_sl = (ostg0, ostg1)
  sem_g = (sem_g0, sem_g1)
  sem_o = (sem_o0, sem_o1)

  def start_gather(i, slot):
    for j in range(SJ):
      pltpu.async_copy(tok_hbm.at[xidx_v.at[i, j]],
                       rows_sl[slot].at[pl.ds(j * SC_, SC_)], sem_g[slot])

  def wait_gather(i, slot):
    for j in range(SJ):
      pltpu.make_async_copy(tok_hbm.at[xidx_v.at[i, j]],
                            rows_sl[slot].at[pl.ds(j * SC_, SC_)],
                            sem_g[slot]).wait()

  def start_out(i, slot):
    pltpu.async_copy(ostg_sl[slot], out_hbm.at[b0 + i], sem_o[slot])

  def wait_out(i, slot):
    pltpu.make_async_copy(ostg_sl[slot], out_hbm.at[b0 + i],
                          sem_o[slot]).wait()

  def compute(i, slot):
    rows = rows_sl[slot]
    ostg = ostg_sl[slot]

    @plsc.parallel_loop(0, S, unroll=2)
    def rowbody(r):
      m = plsc.load_gather(segv, [jnp.full((L,), i, jnp.int32),
                                  jnp.full((L,), r, jnp.int32)])
      vs = []
      for k in range(KD):
        v = rows[r, pl.ds(k * L, L)] + posv[r, pl.ds(k * L, L)]
        v = v + (m * sd[k] + s0[k])
        vs.append(v)
      t = (vs[0] + vs[1]) + (vs[2] + vs[3])
      u = ((vs[0] * vs[0] + vs[1] * vs[1])
           + (vs[2] * vs[2] + vs[3] * vs[3]))
      mean = jnp.full((L,), jnp.sum(t)) * (1.0 / D)
      ex2 = jnp.full((L,), jnp.sum(u)) * (1.0 / D)
      var = ex2 - mean * mean + _EPS
      iv = plsc.bitcast(var, jnp.int32)
      iv = jnp.full((L,), _MAGIC, jnp.int32) - lax.shift_right_logical(iv, 1)
      y = plsc.bitcast(iv, jnp.float32)
      for _ in range(3):
        y = y * (1.5 - 0.5 * var * y * y)
      for k in range(KD):
        o = (vs[k] - mean) * (y * g[k]) + bt[k]
        ostg[r, pl.ds(k * L, L)] = o

  start_gather(0, 0)

  def iter2(it2, carry):
    for s2 in range(2):
      i = it2 * 2 + s2

      @pl.when(i + 1 < nb)
      def _():
        start_gather(i + 1, 1 - s2)

      wait_gather(i, s2)

      @pl.when(i >= 2)
      def _():
        wait_out(i - 2, s2)

      compute(i, s2)
      start_out(i, s2)
    return carry

  lax.fori_loop(0, nb // 2, iter2, 0)
  wait_out(nb - 2, 0)
  wait_out(nb - 1, 1)


def _tr_body(tokT_hbm, out_hbm, buf0, buf1, obuf0, obuf1, tbuf,
             sem_i0, sem_i1, sem_o0, sem_o1):
  d2, v = tokT_hbm.shape
  ncol_full = v // 128
  tail = v - ncol_full * 128
  wid = lax.axis_index("s") * NC + lax.axis_index("c")
  bufs = (buf0, buf1)
  obufs = (obuf0, obuf1)
  sem_i = (sem_i0, sem_i1)
  sem_o = (sem_o0, sem_o1)
  iota = jnp.arange(L, dtype=jnp.int32)

  def col_of(cl):
    return cl * NW + wid

  def in_src(cl):
    off = pl.multiple_of(col_of(cl) * 128, 128)
    return tokT_hbm.at[:, pl.ds(off, 128)]

  def out_dst(cl):
    off = pl.multiple_of(col_of(cl) * 64, 64)
    return out_hbm.at[pl.ds(off, 64)]

  def start_in(cl, slot):
    for t in range(D // 8):
      pltpu.async_copy(in_src(cl).at[pl.ds(8 * t, 8)],
                       bufs[slot].at[pl.ds(8 * t, 8)], sem_i[slot])

  def wait_in(cl, slot):
    for t in range(D // 8):
      pltpu.make_async_copy(in_src(cl).at[pl.ds(8 * t, 8)],
                            bufs[slot].at[pl.ds(8 * t, 8)],
                            sem_i[slot]).wait()

  def start_out(cl, slot):
    pltpu.async_copy(obufs[slot], out_dst(cl), sem_o[slot])

  def wait_out(cl, slot):
    pltpu.make_async_copy(obufs[slot], out_dst(cl), sem_o[slot]).wait()

  def transpose_col(buf, obuf, ntok):
    @plsc.parallel_loop(0, ntok, unroll=8)
    def ttloop(tt):
      xi = jnp.full((L,), tt, jnp.int32)
      for k in range(KD):
        rowvec = plsc.load_gather(buf, [iota + k * L, xi])
        flat = tt * D + k * L
        obuf[flat // 128, pl.ds(pl.multiple_of(flat % 128, 16), L)] = rowvec

  ncols_w = (ncol_full - wid + NW - 1) // NW

  @pl.when(ncols_w > 0)
  def _():
    start_in(0, 0)

    def citer(c2, carry):
      for s2 in range(2):
        cl = c2 * 2 + s2

        @pl.when(cl < ncols_w)
        def _():
          @pl.when(cl + 1 < ncols_w)
          def _():
            start_in(cl + 1, 1 - s2)
          wait_in(cl, s2)

          @pl.when(cl >= 2)
          def _():
            wait_out(cl - 2, s2)
          transpose_col(bufs[s2], obufs[s2], 128)
          start_out(cl, s2)
      return carry

    lax.fori_loop(0, (ncols_w + 1) // 2, citer, 0)

    @pl.when(ncols_w % 2 == 0)
    def _():
      wait_out(ncols_w - 2, 0)
      wait_out(ncols_w - 1, 1)

    @pl.when(ncols_w % 2 == 1)
    def _():
      wait_out(ncols_w - 2, 1)
      wait_out(ncols_w - 1, 0)

  if tail > 0:
    @pl.when(wid == NW - 1)
    def _():
      pltpu.sync_copy(tokT_hbm.at[:, pl.ds(ncol_full * 128, tail)], tbuf)
      transpose_col(tbuf, obuf0, tail)
      nrow = tail * D // 128
      pltpu.sync_copy(
          obuf0.at[pl.ds(0, nrow)],
          out_hbm.at[pl.ds(ncol_full * 64, nrow)])


def _transpose_table(tok_table):
  v, d = tok_table.shape
  vpad = ((v + 127) // 128) * 128
  mesh = plsc.VectorSubcoreMesh(core_axis_name="c", subcore_axis_name="s")
  fn = pl.kernel(
      _tr_body,
      out_type=jax.ShapeDtypeStruct((vpad * d // 128, 128), jnp.float32),
      mesh=mesh,
      compiler_params=pltpu.CompilerParams(
          needs_layout_passes=False, use_tc_tiling_on_sc=True),
      scratch_types=[
          pltpu.VMEM((D, 128), jnp.float32),
          pltpu.VMEM((D, 128), jnp.float32),
          pltpu.VMEM((D, 128), jnp.float32),
          pltpu.VMEM((D, 128), jnp.float32),
          pltpu.VMEM((D, D), jnp.float32),
          pltpu.SemaphoreType.DMA,
          pltpu.SemaphoreType.DMA,
          pltpu.SemaphoreType.DMA,
          pltpu.SemaphoreType.DMA,
      ],
  )
  out = fn(tok_table.T)
  return out.reshape(vpad, d)


def kernel(x, seg, tok_table, pos_table, seg_table, gamma, beta):
  B, seq = x.shape
  d = tok_table.shape[1]
  tok_lin = _transpose_table(tok_table)
  x3 = x.reshape(B, SJ, SC_)
  segf = jnp.pad(seg.astype(jnp.float32), ((0, 0), (0, 256 - seq)))
  g2 = gamma.reshape(KD, L)
  b2 = beta.reshape(KD, L)
  nb = B // NW

  mesh = plsc.VectorSubcoreMesh(core_axis_name="c", subcore_axis_name="s")
  fn = pl.kernel(
      _body,
      out_type=jax.ShapeDtypeStruct((B, seq, d), jnp.float32),
      mesh=mesh,
      compiler_params=pltpu.CompilerParams(
          needs_layout_passes=False, use_tc_tiling_on_sc=False),
      scratch_types=[
          pltpu.VMEM((nb, SJ, SC_), jnp.int32),
          pltpu.VMEM((nb, 256), jnp.float32),
          pltpu.VMEM((S, d), jnp.float32),
          pltpu.VMEM((2, d), jnp.float32),
          pltpu.VMEM((KD, L), jnp.float32),
          pltpu.VMEM((KD, L), jnp.float32),
          pltpu.VMEM((S, d), jnp.float32),
          pltpu.VMEM((S, d), jnp.float32),
          pltpu.VMEM((S, d), jnp.float32),
          pltpu.VMEM((S, d), jnp.float32),
          pltpu.SemaphoreType.DMA,
          pltpu.SemaphoreType.DMA,
          pltpu.SemaphoreType.DMA,
          pltpu.SemaphoreType.DMA,
      ],
  )
  return fn(x3, segf, tok_lin, pos_table, seg_table, g2, b2)


# speedup vs baseline: 1.2448x; 1.2448x over previous
import jax
import jax.numpy as jnp
from jax import lax
from jax.experimental import pallas as pl
from jax.experimental.pallas import tpu as pltpu
from jax.experimental.pallas import tpu_sc as plsc

NC = 2
NS = 16
NW = NC * NS
L = 16

SJ = 2
SC_ = 100
S = SJ * SC_
D = 64
KD = D // L

_EPS = 1e-5
_MAGIC = 0x5F3759DF

_DNUMS = lax.GatherDimensionNumbers(
    offset_dims=(), collapsed_slice_dims=(0,), start_index_map=(0,))


def _bcast_lane(vec, lane):
  idx = jnp.full((L, 1), lane, jnp.int32)
  return lax.gather(vec, idx, dimension_numbers=_DNUMS, slice_sizes=(1,),
                    mode=lax.GatherScatterMode.PROMISE_IN_BOUNDS)


def _body(x_hbm, seg_hbm, tok_hbm, pos_hbm, segtab_hbm, g_hbm, b_hbm,
          out_hbm, xidx_v, segv, posv, segtab_v, gv, bv,
          rows0, rows1, ostg0, ostg1, sem_g0, sem_g1, sem_o0, sem_o1):
  nb = x_hbm.shape[0] // NW
  wid = lax.axis_index("s") * NC + lax.axis_index("c")
  b0 = wid * nb

  pltpu.sync_copy(pos_hbm, posv)
  pltpu.sync_copy(segtab_hbm, segtab_v)
  pltpu.sync_copy(g_hbm, gv)
  pltpu.sync_copy(b_hbm, bv)
  pltpu.sync_copy(x_hbm.at[pl.ds(b0, nb)], xidx_v)
  pltpu.sync_copy(seg_hbm.at[pl.ds(b0, nb)], segv)

  g = [gv[k] for k in range(KD)]
  bt = [bv[k] for k in range(KD)]
  s0 = [segtab_v[0, pl.ds(k * L, L)] for k in range(KD)]
  sd = [segtab_v[1, pl.ds(k * L, L)] - s0[k] for k in range(KD)]

  rows_sl = (rows0, rows1)
  ostg_sl = (ostg0, ostg1)
  sem_g = (sem_g0, sem_g1)
  sem_o = (sem_o0, sem_o1)

  def start_gather(i, slot):
    for j in range(SJ):
      pltpu.async_copy(tok_hbm.at[xidx_v.at[i, j]],
                       rows_sl[slot].at[pl.ds(j * SC_, SC_)], sem_g[slot])

  def wait_gather(i, slot):
    for j in range(SJ):
      pltpu.make_async_copy(tok_hbm.at[xidx_v.at[i, j]],
                            rows_sl[slot].at[pl.ds(j * SC_, SC_)],
                            sem_g[slot]).wait()

  def start_out(i, slot):
    pltpu.async_copy(ostg_sl[slot], out_hbm.at[b0 + i], sem_o[slot])

  def wait_out(i, slot):
    pltpu.make_async_copy(ostg_sl[slot], out_hbm.at[b0 + i],
                          sem_o[slot]).wait()

  def compute(i, slot):
    rows = rows_sl[slot]
    ostg = ostg_sl[slot]

    @plsc.parallel_loop(0, S, unroll=2)
    def rowbody(r):
      m = plsc.load_gather(segv, [jnp.full((L,), i, jnp.int32),
                                  jnp.full((L,), r, jnp.int32)])
      vs = []
      for k in range(KD):
        v = rows[r, pl.ds(k * L, L)] + posv[r, pl.ds(k * L, L)]
        v = v + (m * sd[k] + s0[k])
        vs.append(v)
      t = (vs[0] + vs[1]) + (vs[2] + vs[3])
      u = ((vs[0] * vs[0] + vs[1] * vs[1])
           + (vs[2] * vs[2] + vs[3] * vs[3]))
      mean = jnp.full((L,), jnp.sum(t)) * (1.0 / D)
      ex2 = jnp.full((L,), jnp.sum(u)) * (1.0 / D)
      var = ex2 - mean * mean + _EPS
      iv = plsc.bitcast(var, jnp.int32)
      iv = jnp.full((L,), _MAGIC, jnp.int32) - lax.shift_right_logical(iv, 1)
      y = plsc.bitcast(iv, jnp.float32)
      for _ in range(3):
        y = y * (1.5 - 0.5 * var * y * y)
      for k in range(KD):
        o = (vs[k] - mean) * (y * g[k]) + bt[k]
        ostg[r, pl.ds(k * L, L)] = o

  start_gather(0, 0)

  def iter2(it2, carry):
    for s2 in range(2):
      i = it2 * 2 + s2

      @pl.when(i + 1 < nb)
      def _():
        start_gather(i + 1, 1 - s2)

      wait_gather(i, s2)

      @pl.when(i >= 2)
      def _():
        wait_out(i - 2, s2)

      compute(i, s2)
      start_out(i, s2)
    return carry

  lax.fori_loop(0, nb // 2, iter2, 0)
  wait_out(nb - 2, 0)
  wait_out(nb - 1, 1)


def kernel(x, seg, tok_table, pos_table, seg_table, gamma, beta):
  B, seq = x.shape
  d = tok_table.shape[1]
  x3 = x.reshape(B, SJ, SC_)
  segf = jnp.pad(seg.astype(jnp.float32), ((0, 0), (0, 256 - seq)))
  g2 = gamma.reshape(KD, L)
  b2 = beta.reshape(KD, L)
  nb = B // NW

  mesh = plsc.VectorSubcoreMesh(core_axis_name="c", subcore_axis_name="s")
  fn = pl.kernel(
      _body,
      out_type=jax.ShapeDtypeStruct((B, seq, d), jnp.float32),
      mesh=mesh,
      compiler_params=pltpu.CompilerParams(
          needs_layout_passes=False, use_tc_tiling_on_sc=False),
      scratch_types=[
          pltpu.VMEM((nb, SJ, SC_), jnp.int32),
          pltpu.VMEM((nb, 256), jnp.float32),
          pltpu.VMEM((S, d), jnp.float32),
          pltpu.VMEM((2, d), jnp.float32),
          pltpu.VMEM((KD, L), jnp.float32),
          pltpu.VMEM((KD, L), jnp.float32),
          pltpu.VMEM((S, d), jnp.float32),
          pltpu.VMEM((S, d), jnp.float32),
          pltpu.VMEM((S, d), jnp.float32),
          pltpu.VMEM((S, d), jnp.float32),
          pltpu.SemaphoreType.DMA,
          pltpu.SemaphoreType.DMA,
          pltpu.SemaphoreType.DMA,
          pltpu.SemaphoreType.DMA,
      ],
  )
  return fn(x3, segf, tok_table, pos_table, seg_table, g2, b2)
